# baseline (speedup 1.0000x reference)
import functools

import jax
import jax.numpy as jnp
from jax import lax
from jax.experimental import pallas as pl
from jax.experimental.pallas import tpu as pltpu
from jax.experimental.pallas import tpu_sc as plsc

_TILE = 128
_NLANE = 16


@functools.cache
def _sc_geometry():
    info = plsc.get_sparse_core_info()
    return info.num_cores, info.num_subcores


@functools.partial(jax.jit, static_argnums=(3, 4, 5))
def _phase1(table_t, sv, ord2d, b_per_w, nc, C):
    V = table_t.shape[1]
    B = sv.shape[0]
    mesh = plsc.VectorSubcoreMesh(core_axis_name="c", subcore_axis_name="s")
    n_batches = b_per_w // _TILE
    groups_per_batch = _TILE // _NLANE

    @functools.partial(
        pl.kernel,
        mesh=mesh,
        out_type=jax.ShapeDtypeStruct((B, _TILE), jnp.float32),
        scratch_types=[
            pltpu.VMEM((b_per_w + 32,), jnp.int32),
            pltpu.VMEM((n_batches, _TILE), jnp.int32),
            pltpu.VMEM((_NLANE, C, _TILE), jnp.float32),
            pltpu.VMEM((_TILE, _TILE), jnp.float32),
            pltpu.SemaphoreType.DMA,
            pltpu.SemaphoreType.DMA,
        ],
        compiler_params=pltpu.CompilerParams(needs_layout_passes=False),
    )
    def k(tab, sv_hbm, ord_hbm, stage, svv, jv, tiles, rows_v, gsem, ssem):
        wid = lax.axis_index("s") * nc + lax.axis_index("c")
        base = wid * b_per_w
        iota = lax.iota(jnp.int32, _NLANE)
        pltpu.sync_copy(sv_hbm.at[pl.ds(base, b_per_w)], svv.at[pl.ds(0, b_per_w)])
        pltpu.sync_copy(ord_hbm.at[pl.ds(wid * n_batches, n_batches)], jv)
        row_halves = [iota + _NLANE * h for h in range(C // _NLANE)]

        @pl.loop(0, n_batches)
        def _(t):
            for gi in range(groups_per_batch):
                goff = t * _TILE + gi * _NLANE
                vvec = svv[pl.ds(goff, _NLANE)]
                tvec = vvec >> 7
                ovec = vvec & 127
                t_sc = [tvec[b] for b in range(_NLANE)]
                lead_sc = [jnp.int32(0)]
                new_sc = [None]
                nf = jnp.int32(1)
                for b in range(1, _NLANE):
                    is_new = t_sc[b] != t_sc[b - 1]
                    new_sc.append(is_new)
                    lead_sc.append(
                        jnp.where(is_new, jnp.int32(b), lead_sc[b - 1])
                    )
                    nf = nf + is_new.astype(jnp.int32)

                off0 = pl.multiple_of(t_sc[0] << 7, _TILE)
                pltpu.async_copy(tab.at[:, pl.ds(off0, _TILE)], tiles.at[0], gsem)
                for b in range(1, _NLANE):
                    @pl.when(new_sc[b])
                    def _():
                        off = pl.multiple_of(t_sc[b] << 7, _TILE)
                        pltpu.async_copy(
                            tab.at[:, pl.ds(off, _TILE)], tiles.at[b], gsem
                        )

                @pl.loop(0, nf)
                def _(i):
                    pltpu.make_async_copy(
                        tab.at[:, pl.ds(0, _TILE)], tiles.at[0], gsem
                    ).wait()

                for b in range(_NLANE):
                    slot = jnp.broadcast_to(lead_sc[b], (_NLANE,))
                    col = jnp.broadcast_to(ovec[b], (_NLANE,))
                    r = jnp.full((_NLANE,), gi * _NLANE + b, jnp.int32)
                    for h, rows in enumerate(row_halves):
                        vals = plsc.load_gather(tiles, [slot, rows, col])
                        plsc.store_scatter(rows_v, [r, iota + _NLANE * h], vals)

            pltpu.async_copy(rows_v, stage.at[jv.at[t]], ssem).wait()

    return k(table_t, sv, ord2d)


@functools.partial(jax.jit, static_argnums=(1, 2, 3))
def _phase2(stage, b_per_w, nc, C):
    B = stage.shape[0]
    mesh = plsc.VectorSubcoreMesh(core_axis_name="c", subcore_axis_name="s")
    n_batches = b_per_w // _TILE

    @functools.partial(
        pl.kernel,
        mesh=mesh,
        out_type=jax.ShapeDtypeStruct((C, B), jnp.float32),
        scratch_types=[
            pltpu.VMEM((_TILE, _TILE), jnp.float32),
            pltpu.VMEM((C, b_per_w), jnp.float32),
            pltpu.SemaphoreType.DMA,
            pltpu.SemaphoreType.DMA,
        ],
        compiler_params=pltpu.CompilerParams(needs_layout_passes=False),
    )
    def k(stage_hbm, out, in2, obuf, isem, osem):
        wid = lax.axis_index("s") * nc + lax.axis_index("c")
        base = wid * b_per_w
        iota = lax.iota(jnp.int32, _NLANE)
        row_halves = [iota + _NLANE * h for h in range(C // _NLANE)]

        @pl.loop(0, n_batches)
        def _(t):
            pltpu.async_copy(
                stage_hbm.at[pl.ds(base + t * _TILE, _TILE), :], in2, isem
            ).wait()
            for j in range(_TILE):
                dst_col = jnp.broadcast_to(t * _TILE + j, (_NLANE,))
                row_j = jnp.full((_NLANE,), j, jnp.int32)
                for h, rows in enumerate(row_halves):
                    vals = plsc.load_gather(in2, [row_j, rows])
                    plsc.store_scatter(obuf, [rows, dst_col], vals)

        for tt in range(n_batches):
            ocol = pl.multiple_of(base + tt * _TILE, _TILE)
            pltpu.async_copy(
                obuf.at[:, pl.ds(tt * _TILE, _TILE)],
                out.at[:, pl.ds(ocol, _TILE)],
                osem,
            ).wait()

    return k(stage)


def kernel(boxes, box_indices):
    nm, nb, two, dims = boxes.shape
    C = two * dims
    B = box_indices.shape[0]
    nc, ns = _sc_geometry()
    nw = nc * ns
    b_per_w = B // nw
    table_t = boxes.reshape(nb, C).T
    idx = box_indices.astype(jnp.int32)
    order = jnp.argsort(idx).astype(jnp.int32)
    sv = jnp.take(idx, order)
    ord2d = order.reshape(B // _TILE, _TILE)
    stage = _phase1(table_t, sv, ord2d, b_per_w, nc, C)
    out_t = _phase2(stage, b_per_w, nc, C)
    return out_t.reshape(nm, two, dims, B).transpose(0, 3, 1, 2)


# speedup vs baseline: 1.0659x; 1.0659x over previous
import functools

import jax
import jax.numpy as jnp
from jax import lax
from jax.experimental import pallas as pl
from jax.experimental.pallas import tpu as pltpu
from jax.experimental.pallas import tpu_sc as plsc

_TILE = 128
_NLANE = 16


@functools.cache
def _sc_geometry():
    info = plsc.get_sparse_core_info()
    return info.num_cores, info.num_subcores


@functools.partial(jax.jit, static_argnums=(3, 4, 5))
def _phase1(table_t, keys, idx, b_per_w, nc, C):
    V = table_t.shape[1]
    B = keys.shape[0]
    mesh = plsc.VectorSubcoreMesh(core_axis_name="c", subcore_axis_name="s")
    n_batches = b_per_w // _TILE
    groups_per_batch = _TILE // _NLANE

    @functools.partial(
        pl.kernel,
        mesh=mesh,
        out_type=jax.ShapeDtypeStruct((B, _TILE), jnp.float32),
        scratch_types=[
            pltpu.VMEM((b_per_w,), jnp.int32),
            pltpu.VMEM((B,), jnp.int32),
            pltpu.VMEM((n_batches, _TILE), jnp.int32),
            pltpu.VMEM((_NLANE, C, _TILE), jnp.float32),
            pltpu.VMEM((_TILE, _TILE), jnp.float32),
            pltpu.SemaphoreType.DMA,
            pltpu.SemaphoreType.DMA,
        ],
        compiler_params=pltpu.CompilerParams(needs_layout_passes=False),
    )
    def k(tab, keys_hbm, idx_hbm, stage, kvv, idxfull, jv, tiles, rows_v, gsem, ssem):
        wid = lax.axis_index("s") * nc + lax.axis_index("c")
        base = wid * b_per_w
        iota = lax.iota(jnp.int32, _NLANE)
        pltpu.sync_copy(keys_hbm.at[pl.ds(base, b_per_w)], kvv)
        pltpu.sync_copy(idx_hbm, idxfull)
        row_halves = [iota + _NLANE * h for h in range(C // _NLANE)]

        @pl.loop(0, n_batches)
        def _(t):
            for gi in range(groups_per_batch):
                goff = t * _TILE + gi * _NLANE
                kvec = kvv[pl.ds(goff, _NLANE)]
                jvec = kvec & 16383
                tvec = kvec >> 14
                vv = plsc.load_gather(idxfull, [jvec])
                ovec = vv & 127
                plsc.store_scatter(
                    jv, [jnp.broadcast_to(t, (_NLANE,)), iota + gi * _NLANE], jvec
                )
                t_sc = [tvec[b] for b in range(_NLANE)]
                lead_sc = [jnp.int32(0)]
                new_sc = [None]
                nf = jnp.int32(1)
                for b in range(1, _NLANE):
                    is_new = t_sc[b] != t_sc[b - 1]
                    new_sc.append(is_new)
                    lead_sc.append(
                        jnp.where(is_new, jnp.int32(b), lead_sc[b - 1])
                    )
                    nf = nf + is_new.astype(jnp.int32)

                off0 = pl.multiple_of(t_sc[0] << 7, _TILE)
                pltpu.async_copy(tab.at[:, pl.ds(off0, _TILE)], tiles.at[0], gsem)
                for b in range(1, _NLANE):
                    @pl.when(new_sc[b])
                    def _():
                        off = pl.multiple_of(t_sc[b] << 7, _TILE)
                        pltpu.async_copy(
                            tab.at[:, pl.ds(off, _TILE)], tiles.at[b], gsem
                        )

                @pl.loop(0, nf)
                def _(i):
                    pltpu.make_async_copy(
                        tab.at[:, pl.ds(0, _TILE)], tiles.at[0], gsem
                    ).wait()

                for b in range(_NLANE):
                    slot = jnp.broadcast_to(lead_sc[b], (_NLANE,))
                    col = jnp.broadcast_to(ovec[b], (_NLANE,))
                    r = jnp.full((_NLANE,), gi * _NLANE + b, jnp.int32)
                    for h, rows in enumerate(row_halves):
                        vals = plsc.load_gather(tiles, [slot, rows, col])
                        plsc.store_scatter(rows_v, [r, iota + _NLANE * h], vals)

            pltpu.async_copy(rows_v, stage.at[jv.at[t]], ssem).wait()

    return k(table_t, keys, idx)


@functools.partial(jax.jit, static_argnums=(1, 2, 3))
def _phase2(stage, b_per_w, nc, C):
    B = stage.shape[0]
    mesh = plsc.VectorSubcoreMesh(core_axis_name="c", subcore_axis_name="s")
    n_batches = b_per_w // _TILE

    @functools.partial(
        pl.kernel,
        mesh=mesh,
        out_type=jax.ShapeDtypeStruct((C, B), jnp.float32),
        scratch_types=[
            pltpu.VMEM((_TILE, _TILE), jnp.float32),
            pltpu.VMEM((C, b_per_w), jnp.float32),
            pltpu.SemaphoreType.DMA,
            pltpu.SemaphoreType.DMA,
        ],
        compiler_params=pltpu.CompilerParams(needs_layout_passes=False),
    )
    def k(stage_hbm, out, in2, obuf, isem, osem):
        wid = lax.axis_index("s") * nc + lax.axis_index("c")
        base = wid * b_per_w
        iota = lax.iota(jnp.int32, _NLANE)
        row_halves = [iota + _NLANE * h for h in range(C // _NLANE)]

        @pl.loop(0, n_batches)
        def _(t):
            pltpu.async_copy(
                stage_hbm.at[pl.ds(base + t * _TILE, _TILE), :], in2, isem
            ).wait()
            for j in range(_TILE):
                dst_col = jnp.broadcast_to(t * _TILE + j, (_NLANE,))
                row_j = jnp.full((_NLANE,), j, jnp.int32)
                for h, rows in enumerate(row_halves):
                    vals = plsc.load_gather(in2, [row_j, rows])
                    plsc.store_scatter(obuf, [rows, dst_col], vals)

        for tt in range(n_batches):
            ocol = pl.multiple_of(base + tt * _TILE, _TILE)
            pltpu.async_copy(
                obuf.at[:, pl.ds(tt * _TILE, _TILE)],
                out.at[:, pl.ds(ocol, _TILE)],
                osem,
            ).wait()

    return k(stage)


def kernel(boxes, box_indices):
    nm, nb, two, dims = boxes.shape
    C = two * dims
    B = box_indices.shape[0]
    nc, ns = _sc_geometry()
    nw = nc * ns
    b_per_w = B // nw
    table_t = boxes.reshape(nb, C).T
    idx = box_indices.astype(jnp.int32)
    keys = jnp.sort(((idx >> 7) << 14) | jnp.arange(B, dtype=jnp.int32))
    stage = _phase1(table_t, keys, idx, b_per_w, nc, C)
    out_t = _phase2(stage, b_per_w, nc, C)
    return out_t.reshape(nm, two, dims, B).transpose(0, 3, 1, 2)


# speedup vs baseline: 1.3563x; 1.2725x over previous
import functools

import jax
import jax.numpy as jnp
from jax import lax
from jax.experimental import pallas as pl
from jax.experimental.pallas import tpu as pltpu
from jax.experimental.pallas import tpu_sc as plsc

_TILE = 128
_NLANE = 16


@functools.cache
def _sc_geometry():
    info = plsc.get_sparse_core_info()
    return info.num_cores, info.num_subcores


@functools.partial(jax.jit, static_argnums=(3, 4, 5))
def _phase1(table_t, keys, idx, b_per_w, nc, C):
    V = table_t.shape[1]
    B = keys.shape[0]
    mesh = plsc.VectorSubcoreMesh(core_axis_name="c", subcore_axis_name="s")
    n_batches = b_per_w // _TILE
    groups_per_batch = _TILE // _NLANE

    @functools.partial(
        pl.kernel,
        mesh=mesh,
        out_type=jax.ShapeDtypeStruct((B, _TILE), jnp.float32),
        scratch_types=[
            pltpu.VMEM((b_per_w,), jnp.int32),
            pltpu.VMEM((B,), jnp.int32),
            pltpu.VMEM((n_batches, _TILE), jnp.int32),
            pltpu.VMEM((_NLANE, C, _TILE), jnp.float32),
            pltpu.VMEM((_TILE, _TILE), jnp.float32),
            pltpu.SemaphoreType.DMA,
            pltpu.SemaphoreType.DMA,
        ],
        compiler_params=pltpu.CompilerParams(needs_layout_passes=False),
    )
    def k(tab, keys_hbm, idx_hbm, stage, kvv, idxfull, jv, tiles, rows_v, gsem, ssem):
        wid = lax.axis_index("s") * nc + lax.axis_index("c")
        base = wid * b_per_w
        iota = lax.iota(jnp.int32, _NLANE)
        pltpu.sync_copy(keys_hbm.at[pl.ds(base, b_per_w)], kvv)
        pltpu.sync_copy(idx_hbm, idxfull)
        row_halves = [iota + _NLANE * h for h in range(C // _NLANE)]

        @pl.loop(0, n_batches)
        def _(t):
            for gi in range(groups_per_batch):
                goff = t * _TILE + gi * _NLANE
                kvec = kvv[pl.ds(goff, _NLANE)]
                jvec = kvec & 16383
                tvec = kvec >> 14
                vv = plsc.load_gather(idxfull, [jvec])
                ovec = vv & 127
                plsc.store_scatter(
                    jv, [jnp.broadcast_to(t, (_NLANE,)), iota + gi * _NLANE], jvec
                )
                t_sc = [tvec[b] for b in range(_NLANE)]
                lead_sc = [jnp.int32(0)]
                new_sc = [None]
                nf = jnp.int32(1)
                for b in range(1, _NLANE):
                    is_new = t_sc[b] != t_sc[b - 1]
                    new_sc.append(is_new)
                    lead_sc.append(
                        jnp.where(is_new, jnp.int32(b), lead_sc[b - 1])
                    )
                    nf = nf + is_new.astype(jnp.int32)

                off0 = pl.multiple_of(t_sc[0] << 7, _TILE)
                pltpu.async_copy(tab.at[:, pl.ds(off0, _TILE)], tiles.at[0], gsem)
                for b in range(1, _NLANE):
                    @pl.when(new_sc[b])
                    def _():
                        off = pl.multiple_of(t_sc[b] << 7, _TILE)
                        pltpu.async_copy(
                            tab.at[:, pl.ds(off, _TILE)], tiles.at[b], gsem
                        )

                @pl.loop(0, nf)
                def _(i):
                    pltpu.make_async_copy(
                        tab.at[:, pl.ds(0, _TILE)], tiles.at[0], gsem
                    ).wait()

                for b in range(_NLANE):
                    slot = jnp.broadcast_to(lead_sc[b], (_NLANE,))
                    col = jnp.broadcast_to(ovec[b], (_NLANE,))
                    r = jnp.full((_NLANE,), gi * _NLANE + b, jnp.int32)
                    for h, rows in enumerate(row_halves):
                        vals = plsc.load_gather(tiles, [slot, rows, col])
                        plsc.store_scatter(rows_v, [r, iota + _NLANE * h], vals)

            pltpu.async_copy(rows_v, stage.at[jv.at[t]], ssem).wait()

    return k(table_t, keys, idx)


@functools.partial(jax.jit, static_argnums=(1, 2, 3))
def _phase2(stage, b_per_w, nc, C):
    B = stage.shape[0]
    mesh = plsc.VectorSubcoreMesh(core_axis_name="c", subcore_axis_name="s")
    n_batches = b_per_w // _TILE

    @functools.partial(
        pl.kernel,
        mesh=mesh,
        out_type=jax.ShapeDtypeStruct((C, B), jnp.float32),
        scratch_types=[
            pltpu.VMEM((_TILE, _TILE), jnp.float32),
            pltpu.VMEM((C, b_per_w), jnp.float32),
            pltpu.SemaphoreType.DMA,
            pltpu.SemaphoreType.DMA,
        ],
        compiler_params=pltpu.CompilerParams(needs_layout_passes=False),
    )
    def k(stage_hbm, out, in2, obuf, isem, osem):
        wid = lax.axis_index("s") * nc + lax.axis_index("c")
        base = wid * b_per_w
        iota = lax.iota(jnp.int32, _NLANE)
        row_halves = [iota + _NLANE * h for h in range(C // _NLANE)]

        @pl.loop(0, n_batches)
        def _(t):
            pltpu.async_copy(
                stage_hbm.at[pl.ds(base + t * _TILE, _TILE), :], in2, isem
            ).wait()
            for j in range(_TILE):
                dst_col = jnp.broadcast_to(t * _TILE + j, (_NLANE,))
                row_j = jnp.full((_NLANE,), j, jnp.int32)
                for h, rows in enumerate(row_halves):
                    vals = plsc.load_gather(in2, [row_j, rows])
                    plsc.store_scatter(obuf, [rows, dst_col], vals)

        for tt in range(n_batches):
            ocol = pl.multiple_of(base + tt * _TILE, _TILE)
            pltpu.async_copy(
                obuf.at[:, pl.ds(tt * _TILE, _TILE)],
                out.at[:, pl.ds(ocol, _TILE)],
                osem,
            ).wait()

    return k(stage)


def kernel(boxes, box_indices):
    nm, nb, two, dims = boxes.shape
    C = two * dims
    B = box_indices.shape[0]
    nc, ns = _sc_geometry()
    nw = nc * ns
    b_per_w = B // nw
    table_t = boxes.reshape(nb, C).T
    idx = box_indices.astype(jnp.int32)
    keys = jnp.sort(((idx >> 7) << 14) | jnp.arange(B, dtype=jnp.int32))
    stage = _phase1(table_t, keys, idx, b_per_w, nc, C)
    return stage[:, :C].reshape(nm, B, two, dims)


# speedup vs baseline: 1.3759x; 1.0145x over previous
import functools

import jax
import jax.numpy as jnp
from jax import lax
from jax.experimental import pallas as pl
from jax.experimental.pallas import tpu as pltpu
from jax.experimental.pallas import tpu_sc as plsc

_TILE = 128
_NLANE = 16


@functools.cache
def _sc_geometry():
    info = plsc.get_sparse_core_info()
    return info.num_cores, info.num_subcores


@functools.partial(jax.jit, static_argnums=(3, 4, 5))
def _phase1(table_t, keys, idx, b_per_w, nc, C):
    V = table_t.shape[1]
    B = keys.shape[0]
    mesh = plsc.VectorSubcoreMesh(core_axis_name="c", subcore_axis_name="s")
    n_batches = b_per_w // _TILE
    groups_per_batch = _TILE // _NLANE

    @functools.partial(
        pl.kernel,
        mesh=mesh,
        out_type=jax.ShapeDtypeStruct((B, _TILE), jnp.float32),
        scratch_types=[
            pltpu.VMEM((b_per_w,), jnp.int32),
            pltpu.VMEM((B,), jnp.int32),
            pltpu.VMEM((n_batches, _TILE), jnp.int32),
            pltpu.VMEM((_NLANE, C, _TILE), jnp.float32),
            pltpu.VMEM((2, _TILE, _TILE), jnp.float32),
            pltpu.SemaphoreType.DMA,
            pltpu.SemaphoreType.DMA,
        ],
        compiler_params=pltpu.CompilerParams(needs_layout_passes=False),
    )
    def k(tab, keys_hbm, idx_hbm, stage, kvv, idxfull, jv, tiles, rows_v, gsem, ssem):
        wid = lax.axis_index("s") * nc + lax.axis_index("c")
        base = wid * b_per_w
        iota = lax.iota(jnp.int32, _NLANE)
        pltpu.sync_copy(keys_hbm.at[pl.ds(base, b_per_w)], kvv)
        pltpu.sync_copy(idx_hbm, idxfull)
        row_halves = [iota + _NLANE * h for h in range(C // _NLANE)]

        @pl.loop(0, n_batches)
        def _(t):
            par = t & 1

            @pl.when(t >= 2)
            def _():
                pltpu.make_async_copy(
                    rows_v.at[0], stage.at[jv.at[0]], ssem
                ).wait()

            for gi in range(groups_per_batch):
                goff = t * _TILE + gi * _NLANE
                kvec = kvv[pl.ds(goff, _NLANE)]
                jvec = kvec & 16383
                tvec = kvec >> 14
                vv = plsc.load_gather(idxfull, [jvec])
                ovec = vv & 127
                plsc.store_scatter(
                    jv, [jnp.broadcast_to(t, (_NLANE,)), iota + gi * _NLANE], jvec
                )
                t_sc = [tvec[b] for b in range(_NLANE)]
                lead_sc = [jnp.int32(0)]
                new_sc = [None]
                nf = jnp.int32(1)
                for b in range(1, _NLANE):
                    is_new = t_sc[b] != t_sc[b - 1]
                    new_sc.append(is_new)
                    lead_sc.append(
                        jnp.where(is_new, jnp.int32(b), lead_sc[b - 1])
                    )
                    nf = nf + is_new.astype(jnp.int32)

                off0 = pl.multiple_of(t_sc[0] << 7, _TILE)
                pltpu.async_copy(tab.at[:, pl.ds(off0, _TILE)], tiles.at[0], gsem)
                for b in range(1, _NLANE):
                    @pl.when(new_sc[b])
                    def _():
                        off = pl.multiple_of(t_sc[b] << 7, _TILE)
                        pltpu.async_copy(
                            tab.at[:, pl.ds(off, _TILE)], tiles.at[b], gsem
                        )

                @pl.loop(0, nf)
                def _(i):
                    pltpu.make_async_copy(
                        tab.at[:, pl.ds(0, _TILE)], tiles.at[0], gsem
                    ).wait()

                parv = jnp.broadcast_to(par, (_NLANE,))
                for b in range(_NLANE):
                    slot = jnp.broadcast_to(lead_sc[b], (_NLANE,))
                    col = jnp.broadcast_to(ovec[b], (_NLANE,))
                    r = jnp.full((_NLANE,), gi * _NLANE + b, jnp.int32)
                    for h, rows in enumerate(row_halves):
                        vals = plsc.load_gather(tiles, [slot, rows, col])
                        plsc.store_scatter(
                            rows_v, [parv, r, iota + _NLANE * h], vals
                        )

            @pl.when(par == 0)
            def _():
                pltpu.async_copy(rows_v.at[0], stage.at[jv.at[t]], ssem)

            @pl.when(par == 1)
            def _():
                pltpu.async_copy(rows_v.at[1], stage.at[jv.at[t]], ssem)

        for _d in range(2):
            pltpu.make_async_copy(rows_v.at[0], stage.at[jv.at[0]], ssem).wait()

    return k(table_t, keys, idx)


@functools.partial(jax.jit, static_argnums=(1, 2, 3))
def _phase2(stage, b_per_w, nc, C):
    B = stage.shape[0]
    mesh = plsc.VectorSubcoreMesh(core_axis_name="c", subcore_axis_name="s")
    n_batches = b_per_w // _TILE

    @functools.partial(
        pl.kernel,
        mesh=mesh,
        out_type=jax.ShapeDtypeStruct((C, B), jnp.float32),
        scratch_types=[
            pltpu.VMEM((_TILE, _TILE), jnp.float32),
            pltpu.VMEM((C, b_per_w), jnp.float32),
            pltpu.SemaphoreType.DMA,
            pltpu.SemaphoreType.DMA,
        ],
        compiler_params=pltpu.CompilerParams(needs_layout_passes=False),
    )
    def k(stage_hbm, out, in2, obuf, isem, osem):
        wid = lax.axis_index("s") * nc + lax.axis_index("c")
        base = wid * b_per_w
        iota = lax.iota(jnp.int32, _NLANE)
        row_halves = [iota + _NLANE * h for h in range(C // _NLANE)]

        @pl.loop(0, n_batches)
        def _(t):
            pltpu.async_copy(
                stage_hbm.at[pl.ds(base + t * _TILE, _TILE), :], in2, isem
            ).wait()
            for j in range(_TILE):
                dst_col = jnp.broadcast_to(t * _TILE + j, (_NLANE,))
                row_j = jnp.full((_NLANE,), j, jnp.int32)
                for h, rows in enumerate(row_halves):
                    vals = plsc.load_gather(in2, [row_j, rows])
                    plsc.store_scatter(obuf, [rows, dst_col], vals)

        for tt in range(n_batches):
            ocol = pl.multiple_of(base + tt * _TILE, _TILE)
            pltpu.async_copy(
                obuf.at[:, pl.ds(tt * _TILE, _TILE)],
                out.at[:, pl.ds(ocol, _TILE)],
                osem,
            ).wait()

    return k(stage)


def kernel(boxes, box_indices):
    nm, nb, two, dims = boxes.shape
    C = two * dims
    B = box_indices.shape[0]
    nc, ns = _sc_geometry()
    nw = nc * ns
    b_per_w = B // nw
    table_t = boxes.reshape(nb, C).T
    idx = box_indices.astype(jnp.int32)
    keys = jnp.sort(((idx >> 7) << 14) | jnp.arange(B, dtype=jnp.int32))
    stage = _phase1(table_t, keys, idx, b_per_w, nc, C)
    return stage[:, :C].reshape(nm, B, two, dims)
